# Initial kernel scaffold; baseline (speedup 1.0000x reference)
import functools

import jax
import jax.numpy as jnp
from jax import lax
from jax.experimental import pallas as pl
from jax.experimental.pallas import tpu as pltpu
from jax.experimental.pallas import tpu_sc as plsc

F32 = jnp.float32
I32 = jnp.int32

NN = 100000
EE = 3200000
GG = 64
GP = 80
NP = 100352
NT32 = NP // 32
NT16 = NP // 16
ROW = 128
TOT_ROWS = 25600
EP = TOT_ROWS * ROW
RPT = TOT_ROWS // 32
RB = 8
ROUNDS = RPT // RB


def _mesh():
    return plsc.VectorSubcoreMesh(core_axis_name="c", subcore_axis_name="s")


def _iota16():
    return lax.iota(I32, 16)


@functools.lru_cache(maxsize=None)
def _build_k1():
    @functools.partial(
        pl.kernel,
        out_type=jax.ShapeDtypeStruct((2, NP), F32),
        mesh=_mesh(),
        scratch_types=[
            pltpu.VMEM((RB, ROW), I32),
            pltpu.VMEM((RB, ROW), I32),
            pltpu.VMEM((RB, ROW), F32),
            pltpu.VMEM_SHARED((NP,), F32),
            pltpu.SemaphoreType.DMA,
            pltpu.SemaphoreType.DMA,
        ],
    )
    def k1(x_h, src_h, dst_h, z1_h, out_h, srcb, dstb, gbuf, agg, gsem, ssem):
        c = lax.axis_index("c")
        s = lax.axis_index("s")
        pltpu.sync_copy(z1_h.at[pl.ds(s * NT16, NT16)],
                        agg.at[pl.ds(s * NT16, NT16)])
        plsc.subcore_barrier()
        row0 = (c * 16 + s) * RPT

        def round_body(r, carry):
            base = row0 + r * RB
            pltpu.sync_copy(src_h.at[pl.ds(base, RB)], srcb)
            pltpu.sync_copy(dst_h.at[pl.ds(base, RB)], dstb)
            gds = [pltpu.async_copy(x_h.at[srcb.at[j]], gbuf.at[j], gsem)
                   for j in range(RB)]
            for d in gds:
                d.wait()
            sds = [pltpu.async_copy(gbuf.at[j], agg.at[dstb.at[j]], ssem,
                                    add=True)
                   for j in range(RB)]
            for d in sds:
                d.wait()
            return carry

        lax.fori_loop(0, ROUNDS, round_body, 0)
        plsc.subcore_barrier()
        pltpu.sync_copy(agg.at[pl.ds(s * NT16, NT16)],
                        out_h.at[c, pl.ds(s * NT16, NT16)])

    return k1


@functools.lru_cache(maxsize=None)
def _build_k2():
    @functools.partial(
        pl.kernel,
        out_type=jax.ShapeDtypeStruct((NP, 8), F32),
        mesh=_mesh(),
        scratch_types=[
            pltpu.VMEM((NT32,), F32),
            pltpu.VMEM((NT32,), F32),
            pltpu.VMEM((NT32,), F32),
            pltpu.VMEM((NT32, 8), F32),
            pltpu.VMEM((8, 16), F32),
            pltpu.VMEM((8, 16), F32),
            pltpu.VMEM((8, 8, 16), F32),
            pltpu.VMEM((8, 16), F32),
        ],
    )
    def k2(x_h, p_h, w1_h, b1_h, w2_h, b2_h, out_h,
           xv, p0v, p1v, h1v, w1v, b1v, w2v, b2v):
        c = lax.axis_index("c")
        s = lax.axis_index("s")
        w = c * 16 + s
        noff = w * NT32
        pltpu.sync_copy(x_h.at[pl.ds(noff, NT32)], xv)
        pltpu.sync_copy(p_h.at[0, pl.ds(noff, NT32)], p0v)
        pltpu.sync_copy(p_h.at[1, pl.ds(noff, NT32)], p1v)
        pltpu.sync_copy(w1_h, w1v)
        pltpu.sync_copy(b1_h, b1v)
        pltpu.sync_copy(w2_h, w2v)
        pltpu.sync_copy(b2_h, b2v)
        it = _iota16()

        def grp(g, carry):
            t = (xv[pl.ds(g * 16, 16)] + p0v[pl.ds(g * 16, 16)]
                 + p1v[pl.ds(g * 16, 16)])
            us = [jnp.maximum(t * w1v[j] + b1v[j], 0.0) for j in range(8)]
            rows = g * 16 + it
            for j in range(8):
                v = b2v[j]
                for k in range(8):
                    v = v + us[k] * w2v[k, j]
                e = jnp.where(v > 0.0, v, jnp.exp(v) - 1.0)
                plsc.store_scatter(h1v, [rows, jnp.full((16,), j, I32)], e)
            return carry

        lax.fori_loop(0, NT32 // 16, grp, 0)
        pltpu.sync_copy(h1v, out_h.at[pl.ds(noff, NT32)])

    return k2


@functools.lru_cache(maxsize=None)
def _build_k3():
    @functools.partial(
        pl.kernel,
        out_type=jax.ShapeDtypeStruct((2, NP, 8), F32),
        mesh=_mesh(),
        scratch_types=[
            pltpu.VMEM((RB, ROW), I32),
            pltpu.VMEM((RB, ROW), I32),
            pltpu.VMEM((RB, ROW, 8), F32),
            pltpu.VMEM_SHARED((NP, 8), F32),
            pltpu.SemaphoreType.DMA,
            pltpu.SemaphoreType.DMA,
        ],
    )
    def k3(h1_h, src_h, dst_h, z8_h, out_h, srcb, dstb, gbuf, agg, gsem, ssem):
        c = lax.axis_index("c")
        s = lax.axis_index("s")
        pltpu.sync_copy(z8_h.at[pl.ds(s * NT16, NT16)],
                        agg.at[pl.ds(s * NT16, NT16)])
        plsc.subcore_barrier()
        row0 = (c * 16 + s) * RPT

        def round_body(r, carry):
            base = row0 + r * RB
            pltpu.sync_copy(src_h.at[pl.ds(base, RB)], srcb)
            pltpu.sync_copy(dst_h.at[pl.ds(base, RB)], dstb)
            gds = [pltpu.async_copy(h1_h.at[srcb.at[j]], gbuf.at[j], gsem)
                   for j in range(RB)]
            for d in gds:
                d.wait()
            sds = [pltpu.async_copy(gbuf.at[j], agg.at[dstb.at[j]], ssem,
                                    add=True)
                   for j in range(RB)]
            for d in sds:
                d.wait()
            return carry

        lax.fori_loop(0, ROUNDS, round_body, 0)
        plsc.subcore_barrier()
        pltpu.sync_copy(agg.at[pl.ds(s * NT16, NT16)],
                        out_h.at[c, pl.ds(s * NT16, NT16)])

    return k3


@functools.lru_cache(maxsize=None)
def _build_k4():
    @functools.partial(
        pl.kernel,
        out_type=(jax.ShapeDtypeStruct((32, GP * 8), F32),
                  jax.ShapeDtypeStruct((32, GP), F32)),
        mesh=_mesh(),
        scratch_types=[
            pltpu.VMEM((NT32, 8), F32),
            pltpu.VMEM((NT32, 8), F32),
            pltpu.VMEM((NT32, 8), F32),
            pltpu.VMEM((NT32,), I32),
            pltpu.VMEM((GP * 8,), F32),
            pltpu.VMEM((GP,), F32),
            pltpu.VMEM((8, 16), F32),
            pltpu.VMEM((8, 16), F32),
            pltpu.VMEM((8, 8, 16), F32),
            pltpu.VMEM((8, 16), F32),
        ],
    )
    def k4(h1_h, p_h, batch_h, w3_h, b3_h, w4_h, b4_h, outp_h, outc_h,
           h1v, p0v, p1v, bv, poolv, cntv, w3v, b3v, w4v, b4v):
        c = lax.axis_index("c")
        s = lax.axis_index("s")
        w = c * 16 + s
        noff = w * NT32
        pltpu.sync_copy(h1_h.at[pl.ds(noff, NT32)], h1v)
        pltpu.sync_copy(p_h.at[0, pl.ds(noff, NT32)], p0v)
        pltpu.sync_copy(p_h.at[1, pl.ds(noff, NT32)], p1v)
        pltpu.sync_copy(batch_h.at[pl.ds(noff, NT32)], bv)
        pltpu.sync_copy(w3_h, w3v)
        pltpu.sync_copy(b3_h, b3v)
        pltpu.sync_copy(w4_h, w4v)
        pltpu.sync_copy(b4_h, b4v)
        it = _iota16()
        z16 = jnp.zeros((16,), F32)
        ones16 = jnp.ones((16,), F32)

        def zero_pool(i, carry):
            poolv[pl.ds(i * 16, 16)] = z16
            return carry

        lax.fori_loop(0, GP * 8 // 16, zero_pool, 0)
        for i in range(GP // 16):
            cntv[pl.ds(i * 16, 16)] = z16

        def grp(g, carry):
            rows = g * 16 + it
            ts = []
            for k in range(8):
                ck = jnp.full((16,), k, I32)
                tk = (plsc.load_gather(h1v, [rows, ck])
                      + plsc.load_gather(p0v, [rows, ck])
                      + plsc.load_gather(p1v, [rows, ck]))
                ts.append(tk)
            us = []
            for j in range(8):
                u = b3v[j]
                for k in range(8):
                    u = u + ts[k] * w3v[k, j]
                us.append(jnp.maximum(u, 0.0))
            b16 = bv[pl.ds(g * 16, 16)]
            b8 = b16 * 8
            for j in range(8):
                v = b4v[j]
                for k in range(8):
                    v = v + us[k] * w4v[k, j]
                plsc.addupdate_scatter(poolv, [b8 + j], v)
            plsc.addupdate_scatter(cntv, [b16], ones16)
            return carry

        lax.fori_loop(0, NT32 // 16, grp, 0)
        pltpu.sync_copy(poolv, outp_h.at[w])
        pltpu.sync_copy(cntv, outc_h.at[w])

    return k4


@functools.lru_cache(maxsize=None)
def _build_k5():
    @functools.partial(
        pl.kernel,
        out_type=jax.ShapeDtypeStruct((GG,), F32),
        mesh=_mesh(),
        scratch_types=[
            pltpu.VMEM((32, GP * 8), F32),
            pltpu.VMEM((32, GP), F32),
            pltpu.VMEM((GP * 8,), F32),
            pltpu.VMEM((GP,), F32),
            pltpu.VMEM((8, 16), F32),
            pltpu.VMEM((16,), F32),
            pltpu.VMEM((GG,), F32),
        ],
    )
    def k5(pp_h, pc_h, wfc_h, bfc_h, out_h, ppv, pcv, accp, accc, wfcv,
           bfcv, outv):
        c = lax.axis_index("c")
        s = lax.axis_index("s")

        @pl.when(jnp.logical_and(c == 0, s == 0))
        def _():
            pltpu.sync_copy(pp_h, ppv)
            pltpu.sync_copy(pc_h, pcv)
            pltpu.sync_copy(wfc_h, wfcv)
            pltpu.sync_copy(bfc_h, bfcv)
            it = _iota16()

            def accp_body(i, carry):
                acc = ppv[0, pl.ds(i * 16, 16)]
                for w in range(1, 32):
                    acc = acc + ppv[w, pl.ds(i * 16, 16)]
                accp[pl.ds(i * 16, 16)] = acc
                return carry

            lax.fori_loop(0, GP * 8 // 16, accp_body, 0)
            for i in range(GP // 16):
                acc = pcv[0, pl.ds(i * 16, 16)]
                for w in range(1, 32):
                    acc = acc + pcv[w, pl.ds(i * 16, 16)]
                accc[pl.ds(i * 16, 16)] = acc
            for gg in range(GG // 16):
                cnt = jnp.maximum(accc[pl.ds(gg * 16, 16)], 1.0)
                gidx = (gg * 16 + it) * 8
                z = bfcv[pl.ds(0, 16)]
                for f in range(8):
                    mf = plsc.load_gather(accp, [gidx + f]) / cnt
                    z = z + mf * wfcv[f]
                sig = 1.0 / (1.0 + jnp.exp(-z))
                outv[pl.ds(gg * 16, 16)] = sig
            pltpu.sync_copy(outv, out_h)

    return k5


def kernel(x, edge_index, batch, W1, b1, W2, b2, W3, b3, W4, b4, Wfc, bfc):
    xf = x.reshape(-1).astype(F32)
    xp = jnp.concatenate([xf, jnp.zeros((NP - NN,), F32)])
    src = edge_index[0].astype(I32)
    dst = edge_index[1].astype(I32)
    srcp = jnp.concatenate([src, jnp.zeros((EP - EE,), I32)]).reshape(
        TOT_ROWS, ROW)
    dstp = jnp.concatenate([dst, jnp.full((EP - EE,), NN, I32)]).reshape(
        TOT_ROWS, ROW)
    batchp = jnp.concatenate([batch.astype(I32),
                              jnp.full((NP - NN,), GG, I32)])
    z1 = jnp.zeros((NP,), F32)
    z8 = jnp.zeros((NP, 8), F32)
    w1b = jnp.broadcast_to(W1.reshape(8, 1), (8, 16))
    b1b = jnp.broadcast_to(b1.reshape(8, 1), (8, 16))
    w2b = jnp.broadcast_to(W2.reshape(8, 8, 1), (8, 8, 16))
    b2b = jnp.broadcast_to(b2.reshape(8, 1), (8, 16))
    w3b = jnp.broadcast_to(W3.reshape(8, 8, 1), (8, 8, 16))
    b3b = jnp.broadcast_to(b3.reshape(8, 1), (8, 16))
    w4b = jnp.broadcast_to(W4.reshape(8, 8, 1), (8, 8, 16))
    b4b = jnp.broadcast_to(b4.reshape(8, 1), (8, 16))
    wfcb = jnp.broadcast_to(Wfc.reshape(8, 1), (8, 16))
    bfcb = jnp.broadcast_to(bfc.reshape(1), (16,))

    p1 = _build_k1()(xp, srcp, dstp, z1)
    h1 = _build_k2()(xp, p1, w1b, b1b, w2b, b2b)
    p2 = _build_k3()(h1, srcp, dstp, z8)
    pooled_p, cnt_p = _build_k4()(h1, p2, batchp, w3b, b3b, w4b, b4b)
    out = _build_k5()(pooled_p, cnt_p, wfcb, bfcb)
    return out


# speedup vs baseline: 35.6373x; 35.6373x over previous
import functools

import jax
import jax.numpy as jnp
from jax import lax
from jax.experimental import pallas as pl
from jax.experimental.pallas import tpu as pltpu
from jax.experimental.pallas import tpu_sc as plsc

F32 = jnp.float32
I32 = jnp.int32

NN = 100000
EE = 3200000
GG = 64
GP = 80
CW = 128
NP = 102400
NT32 = NP // 32
NT16 = NP // 16
ROW = 128
TOT_ROWS = 25600
EP = TOT_ROWS * ROW
RPT = TOT_ROWS // 32
RB = 8
ROUNDS = RPT // RB


def _mesh():
    return plsc.VectorSubcoreMesh(core_axis_name="c", subcore_axis_name="s")


def _params():
    return pltpu.CompilerParams(needs_layout_passes=False,
                                use_tc_tiling_on_sc=False)


def _iota16():
    return lax.iota(I32, 16)


@functools.lru_cache(maxsize=None)
def _build_k1():
    @functools.partial(
        pl.kernel,
        out_type=(jax.ShapeDtypeStruct((NP,), F32),
                  jax.ShapeDtypeStruct((NP,), F32)),
        mesh=_mesh(),
        compiler_params=_params(),
        scratch_types=[
            pltpu.VMEM((RB, ROW), I32),
            pltpu.VMEM((RB, ROW), I32),
            pltpu.VMEM((RB, ROW), F32),
            pltpu.VMEM_SHARED((NP,), F32),
            pltpu.SemaphoreType.DMA,
            pltpu.SemaphoreType.DMA,
        ],
    )
    def k1(x_h, src_h, dst_h, z1_h, out0_h, out1_h,
           srcb, dstb, gbuf, agg, gsem, ssem):
        c = lax.axis_index("c")
        s = lax.axis_index("s")
        pltpu.sync_copy(z1_h.at[pl.ds(s * NT16, NT16)],
                        agg.at[pl.ds(s * NT16, NT16)])
        plsc.subcore_barrier()
        row0 = (c * 16 + s) * RPT

        def round_body(r, carry):
            base = row0 + r * RB
            pltpu.sync_copy(src_h.at[pl.ds(base, RB)], srcb)
            pltpu.sync_copy(dst_h.at[pl.ds(base, RB)], dstb)
            gds = [pltpu.async_copy(x_h.at[srcb.at[j]], gbuf.at[j], gsem)
                   for j in range(RB)]
            for d in gds:
                d.wait()
            sds = [pltpu.async_copy(gbuf.at[j], agg.at[dstb.at[j]], ssem,
                                    add=True)
                   for j in range(RB)]
            for d in sds:
                d.wait()
            return carry

        lax.fori_loop(0, ROUNDS, round_body, 0)
        plsc.subcore_barrier()
        sl = pl.ds(s * NT16, NT16)

        @pl.when(c == 0)
        def _():
            pltpu.sync_copy(agg.at[sl], out0_h.at[sl])

        @pl.when(c == 1)
        def _():
            pltpu.sync_copy(agg.at[sl], out1_h.at[sl])

    return k1


@functools.lru_cache(maxsize=None)
def _build_k2():
    @functools.partial(
        pl.kernel,
        out_type=jax.ShapeDtypeStruct((NP * 8,), F32),
        mesh=_mesh(),
        compiler_params=_params(),
        scratch_types=[
            pltpu.VMEM((NT32,), F32),
            pltpu.VMEM((NT32,), F32),
            pltpu.VMEM((NT32,), F32),
            pltpu.VMEM((NT32 * 8,), F32),
            pltpu.VMEM((8, 16), F32),
            pltpu.VMEM((8, 16), F32),
            pltpu.VMEM((8, 8, 16), F32),
            pltpu.VMEM((8, 16), F32),
        ],
    )
    def k2(x_h, p0_h, p1_h, w1_h, b1_h, w2_h, b2_h, out_h,
           xv, p0v, p1v, h1v, w1v, b1v, w2v, b2v):
        c = lax.axis_index("c")
        s = lax.axis_index("s")
        w = c * 16 + s
        noff = w * NT32
        pltpu.sync_copy(x_h.at[pl.ds(noff, NT32)], xv)
        pltpu.sync_copy(p0_h.at[pl.ds(noff, NT32)], p0v)
        pltpu.sync_copy(p1_h.at[pl.ds(noff, NT32)], p1v)
        pltpu.sync_copy(w1_h, w1v)
        pltpu.sync_copy(b1_h, b1v)
        pltpu.sync_copy(w2_h, w2v)
        pltpu.sync_copy(b2_h, b2v)
        it = _iota16()

        def grp(g, carry):
            t = (xv[pl.ds(g * 16, 16)] + p0v[pl.ds(g * 16, 16)]
                 + p1v[pl.ds(g * 16, 16)])
            us = [jnp.maximum(t * w1v[j] + b1v[j], 0.0) for j in range(8)]
            rows8 = (g * 16 + it) * 8
            for j in range(8):
                v = b2v[j]
                for k in range(8):
                    v = v + us[k] * w2v[k, j]
                e = jnp.where(v > 0.0, v, jnp.exp(v) - 1.0)
                plsc.store_scatter(h1v, [rows8 + j], e)
            return carry

        lax.fori_loop(0, NT32 // 16, grp, 0)
        pltpu.sync_copy(h1v, out_h.at[pl.ds(noff * 8, NT32 * 8)])

    return k2


@functools.lru_cache(maxsize=None)
def _build_k3():
    @functools.partial(
        pl.kernel,
        out_type=(jax.ShapeDtypeStruct((NP, 8), F32),
                  jax.ShapeDtypeStruct((NP, 8), F32)),
        mesh=_mesh(),
        compiler_params=_params(),
        scratch_types=[
            pltpu.VMEM((RB, ROW), I32),
            pltpu.VMEM((RB, ROW), I32),
            pltpu.VMEM((RB, ROW, 8), F32),
            pltpu.VMEM_SHARED((NP, 8), F32),
            pltpu.SemaphoreType.DMA,
            pltpu.SemaphoreType.DMA,
        ],
    )
    def k3(h1_h, src_h, dst_h, z8_h, out0_h, out1_h,
           srcb, dstb, gbuf, agg, gsem, ssem):
        c = lax.axis_index("c")
        s = lax.axis_index("s")
        pltpu.sync_copy(z8_h.at[pl.ds(s * NT16, NT16)],
                        agg.at[pl.ds(s * NT16, NT16)])
        plsc.subcore_barrier()
        row0 = (c * 16 + s) * RPT

        def round_body(r, carry):
            base = row0 + r * RB
            pltpu.sync_copy(src_h.at[pl.ds(base, RB)], srcb)
            pltpu.sync_copy(dst_h.at[pl.ds(base, RB)], dstb)
            gds = [pltpu.async_copy(h1_h.at[srcb.at[j]], gbuf.at[j], gsem)
                   for j in range(RB)]
            for d in gds:
                d.wait()
            sds = [pltpu.async_copy(gbuf.at[j], agg.at[dstb.at[j]], ssem,
                                    add=True)
                   for j in range(RB)]
            for d in sds:
                d.wait()
            return carry

        lax.fori_loop(0, ROUNDS, round_body, 0)
        plsc.subcore_barrier()
        sl = pl.ds(s * NT16, NT16)

        @pl.when(c == 0)
        def _():
            pltpu.sync_copy(agg.at[sl], out0_h.at[sl])

        @pl.when(c == 1)
        def _():
            pltpu.sync_copy(agg.at[sl], out1_h.at[sl])

    return k3


@functools.lru_cache(maxsize=None)
def _build_k4():
    @functools.partial(
        pl.kernel,
        out_type=(jax.ShapeDtypeStruct((32 * GP * 8,), F32),
                  jax.ShapeDtypeStruct((32 * CW,), F32)),
        mesh=_mesh(),
        compiler_params=_params(),
        scratch_types=[
            pltpu.VMEM((NT32 * 8,), F32),
            pltpu.VMEM((NT32 * 8,), F32),
            pltpu.VMEM((NT32 * 8,), F32),
            pltpu.VMEM((NT32,), I32),
            pltpu.VMEM((GP * 8,), F32),
            pltpu.VMEM((CW,), F32),
            pltpu.VMEM((8, 8, 16), F32),
            pltpu.VMEM((8, 16), F32),
            pltpu.VMEM((8, 8, 16), F32),
            pltpu.VMEM((8, 16), F32),
        ],
    )
    def k4(h1_h, p0_h, p1_h, batch_h, w3_h, b3_h, w4_h, b4_h,
           outp_h, outc_h,
           h1v, p0v, p1v, bv, poolv, cntv, w3v, b3v, w4v, b4v):
        c = lax.axis_index("c")
        s = lax.axis_index("s")
        w = c * 16 + s
        noff = w * NT32
        pltpu.sync_copy(h1_h.at[pl.ds(noff * 8, NT32 * 8)], h1v)
        pltpu.sync_copy(p0_h.at[pl.ds(noff * 8, NT32 * 8)], p0v)
        pltpu.sync_copy(p1_h.at[pl.ds(noff * 8, NT32 * 8)], p1v)
        pltpu.sync_copy(batch_h.at[pl.ds(noff, NT32)], bv)
        pltpu.sync_copy(w3_h, w3v)
        pltpu.sync_copy(b3_h, b3v)
        pltpu.sync_copy(w4_h, w4v)
        pltpu.sync_copy(b4_h, b4v)
        it = _iota16()
        z16 = jnp.zeros((16,), F32)
        ones16 = jnp.ones((16,), F32)

        def zero_pool(i, carry):
            poolv[pl.ds(i * 16, 16)] = z16
            return carry

        lax.fori_loop(0, GP * 8 // 16, zero_pool, 0)
        for i in range(CW // 16):
            cntv[pl.ds(i * 16, 16)] = z16

        def grp(g, carry):
            rows8 = (g * 16 + it) * 8
            ts = []
            for k in range(8):
                tk = (plsc.load_gather(h1v, [rows8 + k])
                      + plsc.load_gather(p0v, [rows8 + k])
                      + plsc.load_gather(p1v, [rows8 + k]))
                ts.append(tk)
            us = []
            for j in range(8):
                u = b3v[j]
                for k in range(8):
                    u = u + ts[k] * w3v[k, j]
                us.append(jnp.maximum(u, 0.0))
            b16 = bv[pl.ds(g * 16, 16)]
            b8 = b16 * 8
            for j in range(8):
                v = b4v[j]
                for k in range(8):
                    v = v + us[k] * w4v[k, j]
                plsc.addupdate_scatter(poolv, [b8 + j], v)
            plsc.addupdate_scatter(cntv, [b16], ones16)
            return carry

        lax.fori_loop(0, NT32 // 16, grp, 0)
        pltpu.sync_copy(poolv, outp_h.at[pl.ds(w * GP * 8, GP * 8)])
        pltpu.sync_copy(cntv, outc_h.at[pl.ds(w * CW, CW)])

    return k4


@functools.lru_cache(maxsize=None)
def _build_k5():
    @functools.partial(
        pl.kernel,
        out_type=jax.ShapeDtypeStruct((GG,), F32),
        mesh=_mesh(),
        compiler_params=_params(),
        scratch_types=[
            pltpu.VMEM((32 * GP * 8,), F32),
            pltpu.VMEM((32 * CW,), F32),
            pltpu.VMEM((GP * 8,), F32),
            pltpu.VMEM((CW,), F32),
            pltpu.VMEM((8, 16), F32),
            pltpu.VMEM((16,), F32),
            pltpu.VMEM((GG,), F32),
        ],
    )
    def k5(pp_h, pc_h, wfc_h, bfc_h, out_h, ppv, pcv, accp, accc, wfcv,
           bfcv, outv):
        c = lax.axis_index("c")
        s = lax.axis_index("s")

        @pl.when(jnp.logical_and(c == 0, s == 0))
        def _():
            pltpu.sync_copy(pp_h, ppv)
            pltpu.sync_copy(pc_h, pcv)
            pltpu.sync_copy(wfc_h, wfcv)
            pltpu.sync_copy(bfc_h, bfcv)
            it = _iota16()

            def accp_body(i, carry):
                acc = ppv[pl.ds(i * 16, 16)]
                for w in range(1, 32):
                    acc = acc + ppv[pl.ds(w * GP * 8 + i * 16, 16)]
                accp[pl.ds(i * 16, 16)] = acc
                return carry

            lax.fori_loop(0, GP * 8 // 16, accp_body, 0)
            for i in range(GP // 16):
                acc = pcv[pl.ds(i * 16, 16)]
                for w in range(1, 32):
                    acc = acc + pcv[pl.ds(w * CW + i * 16, 16)]
                accc[pl.ds(i * 16, 16)] = acc
            for gg in range(GG // 16):
                cnt = jnp.maximum(accc[pl.ds(gg * 16, 16)], 1.0)
                gidx = (gg * 16 + it) * 8
                z = bfcv[pl.ds(0, 16)]
                for f in range(8):
                    mf = plsc.load_gather(accp, [gidx + f]) / cnt
                    z = z + mf * wfcv[f]
                sig = 1.0 / (1.0 + jnp.exp(-z))
                outv[pl.ds(gg * 16, 16)] = sig
            pltpu.sync_copy(outv, out_h)

    return k5


def kernel(x, edge_index, batch, W1, b1, W2, b2, W3, b3, W4, b4, Wfc, bfc):
    xf = x.reshape(-1).astype(F32)
    xp = jnp.concatenate([xf, jnp.zeros((NP - NN,), F32)])
    src = edge_index[0].astype(I32)
    dst = edge_index[1].astype(I32)
    srcp = jnp.concatenate([src, jnp.zeros((EP - EE,), I32)]).reshape(
        TOT_ROWS, ROW)
    dstp = jnp.concatenate([dst, jnp.full((EP - EE,), NN, I32)]).reshape(
        TOT_ROWS, ROW)
    batchp = jnp.concatenate([batch.astype(I32),
                              jnp.full((NP - NN,), GG, I32)])
    z1 = jnp.zeros((NP,), F32)
    z8 = jnp.zeros((NP, 8), F32)
    w1b = jnp.broadcast_to(W1.reshape(8, 1), (8, 16))
    b1b = jnp.broadcast_to(b1.reshape(8, 1), (8, 16))
    w2b = jnp.broadcast_to(W2.reshape(8, 8, 1), (8, 8, 16))
    b2b = jnp.broadcast_to(b2.reshape(8, 1), (8, 16))
    w3b = jnp.broadcast_to(W3.reshape(8, 8, 1), (8, 8, 16))
    b3b = jnp.broadcast_to(b3.reshape(8, 1), (8, 16))
    w4b = jnp.broadcast_to(W4.reshape(8, 8, 1), (8, 8, 16))
    b4b = jnp.broadcast_to(b4.reshape(8, 1), (8, 16))
    wfcb = jnp.broadcast_to(Wfc.reshape(8, 1), (8, 16))
    bfcb = jnp.broadcast_to(bfc.reshape(1), (16,))

    p1a, p1b = _build_k1()(xp, srcp, dstp, z1)
    h1f = _build_k2()(xp, p1a, p1b, w1b, b1b, w2b, b2b)
    p2a, p2b = _build_k3()(h1f.reshape(NP, 8), srcp, dstp, z8)
    pooled_p, cnt_p = _build_k4()(h1f, p2a.reshape(-1), p2b.reshape(-1),
                                  batchp, w3b, b3b, w4b, b4b)
    out = _build_k5()(pooled_p, cnt_p, wfcb, bfcb)
    return out


# speedup vs baseline: 44.5984x; 1.2515x over previous
import functools

import jax
import jax.numpy as jnp
from jax import lax
from jax.experimental import pallas as pl
from jax.experimental.pallas import tpu as pltpu
from jax.experimental.pallas import tpu_sc as plsc

F32 = jnp.float32
I32 = jnp.int32

NN = 100000
EE = 3200000
GG = 64
GP = 80
CW = 128
NP = 102400
NT32 = NP // 32
NT16 = NP // 16
EP = 3276800
EPT = EP // 32
CH = 2048
NRND = EPT // CH
PAIRS = NRND // 2


def _mesh():
    return plsc.VectorSubcoreMesh(core_axis_name="c", subcore_axis_name="s")


def _params():
    return pltpu.CompilerParams(needs_layout_passes=False,
                                use_tc_tiling_on_sc=False)


def _iota16():
    return lax.iota(I32, 16)


@functools.lru_cache(maxsize=None)
def _build_edge_phase(width):
    tshape = (NP,) if width == 1 else (NP, width)
    bshape = (CH,) if width == 1 else (CH, width)

    @functools.partial(
        pl.kernel,
        out_type=(jax.ShapeDtypeStruct(tshape, F32),
                  jax.ShapeDtypeStruct(tshape, F32)),
        mesh=_mesh(),
        compiler_params=_params(),
        scratch_types=[
            pltpu.VMEM((CH,), I32),
            pltpu.VMEM((CH,), I32),
            pltpu.VMEM(bshape, F32),
            pltpu.VMEM((CH,), I32),
            pltpu.VMEM((CH,), I32),
            pltpu.VMEM(bshape, F32),
            pltpu.VMEM_SHARED(tshape, F32),
            pltpu.SemaphoreType.DMA,
            pltpu.SemaphoreType.DMA,
            pltpu.SemaphoreType.DMA,
            pltpu.SemaphoreType.DMA,
        ],
    )
    def kedge(tab_h, src_h, dst_h, z_h, out0_h, out1_h,
              srcbA, dstbA, gbufA, srcbB, dstbB, gbufB, agg,
              gsemA, ssemA, gsemB, ssemB):
        c = lax.axis_index("c")
        s = lax.axis_index("s")
        pltpu.sync_copy(z_h.at[pl.ds(s * NT16, NT16)],
                        agg.at[pl.ds(s * NT16, NT16)])
        plsc.subcore_barrier()
        e0 = (c * 16 + s) * EPT

        def pair_body(i, carry):
            baseA = e0 + (2 * i) * CH
            baseB = baseA + CH

            @pl.when(i > 0)
            def _():
                pltpu.make_async_copy(gbufA, agg.at[dstbA], ssemA).wait()

            pltpu.sync_copy(src_h.at[pl.ds(baseA, CH)], srcbA)
            pltpu.sync_copy(dst_h.at[pl.ds(baseA, CH)], dstbA)
            gdA = pltpu.async_copy(tab_h.at[srcbA], gbufA, gsemA)

            @pl.when(i > 0)
            def _():
                pltpu.make_async_copy(gbufB, agg.at[dstbB], ssemB).wait()

            pltpu.sync_copy(src_h.at[pl.ds(baseB, CH)], srcbB)
            pltpu.sync_copy(dst_h.at[pl.ds(baseB, CH)], dstbB)
            gdA.wait()
            pltpu.async_copy(gbufA, agg.at[dstbA], ssemA, add=True)
            gdB = pltpu.async_copy(tab_h.at[srcbB], gbufB, gsemB)
            gdB.wait()
            pltpu.async_copy(gbufB, agg.at[dstbB], ssemB, add=True)
            return carry

        lax.fori_loop(0, PAIRS, pair_body, 0)
        pltpu.make_async_copy(gbufA, agg.at[dstbA], ssemA).wait()
        pltpu.make_async_copy(gbufB, agg.at[dstbB], ssemB).wait()
        plsc.subcore_barrier()
        sl = pl.ds(s * NT16, NT16)

        @pl.when(c == 0)
        def _():
            pltpu.sync_copy(agg.at[sl], out0_h.at[sl])

        @pl.when(c == 1)
        def _():
            pltpu.sync_copy(agg.at[sl], out1_h.at[sl])

    return kedge


@functools.lru_cache(maxsize=None)
def _build_k2():
    @functools.partial(
        pl.kernel,
        out_type=jax.ShapeDtypeStruct((NP * 8,), F32),
        mesh=_mesh(),
        compiler_params=_params(),
        scratch_types=[
            pltpu.VMEM((NT32,), F32),
            pltpu.VMEM((NT32,), F32),
            pltpu.VMEM((NT32,), F32),
            pltpu.VMEM((NT32 * 8,), F32),
            pltpu.VMEM((8, 16), F32),
            pltpu.VMEM((8, 16), F32),
            pltpu.VMEM((8, 8, 16), F32),
            pltpu.VMEM((8, 16), F32),
        ],
    )
    def k2(x_h, p0_h, p1_h, w1_h, b1_h, w2_h, b2_h, out_h,
           xv, p0v, p1v, h1v, w1v, b1v, w2v, b2v):
        c = lax.axis_index("c")
        s = lax.axis_index("s")
        w = c * 16 + s
        noff = w * NT32
        pltpu.sync_copy(x_h.at[pl.ds(noff, NT32)], xv)
        pltpu.sync_copy(p0_h.at[pl.ds(noff, NT32)], p0v)
        pltpu.sync_copy(p1_h.at[pl.ds(noff, NT32)], p1v)
        pltpu.sync_copy(w1_h, w1v)
        pltpu.sync_copy(b1_h, b1v)
        pltpu.sync_copy(w2_h, w2v)
        pltpu.sync_copy(b2_h, b2v)
        it = _iota16()

        def grp(g, carry):
            t = (xv[pl.ds(g * 16, 16)] + p0v[pl.ds(g * 16, 16)]
                 + p1v[pl.ds(g * 16, 16)])
            us = [jnp.maximum(t * w1v[j] + b1v[j], 0.0) for j in range(8)]
            rows8 = (g * 16 + it) * 8
            for j in range(8):
                v = b2v[j]
                for k in range(8):
                    v = v + us[k] * w2v[k, j]
                e = jnp.where(v > 0.0, v, jnp.exp(v) - 1.0)
                plsc.store_scatter(h1v, [rows8 + j], e)
            return carry

        lax.fori_loop(0, NT32 // 16, grp, 0)
        pltpu.sync_copy(h1v, out_h.at[pl.ds(noff * 8, NT32 * 8)])

    return k2


@functools.lru_cache(maxsize=None)
def _build_k4():
    @functools.partial(
        pl.kernel,
        out_type=(jax.ShapeDtypeStruct((32 * GP * 8,), F32),
                  jax.ShapeDtypeStruct((32 * CW,), F32)),
        mesh=_mesh(),
        compiler_params=_params(),
        scratch_types=[
            pltpu.VMEM((NT32 * 8,), F32),
            pltpu.VMEM((NT32 * 8,), F32),
            pltpu.VMEM((NT32 * 8,), F32),
            pltpu.VMEM((NT32,), I32),
            pltpu.VMEM((GP * 8,), F32),
            pltpu.VMEM((CW,), F32),
            pltpu.VMEM((8, 8, 16), F32),
            pltpu.VMEM((8, 16), F32),
            pltpu.VMEM((8, 8, 16), F32),
            pltpu.VMEM((8, 16), F32),
        ],
    )
    def k4(h1_h, p0_h, p1_h, batch_h, w3_h, b3_h, w4_h, b4_h,
           outp_h, outc_h,
           h1v, p0v, p1v, bv, poolv, cntv, w3v, b3v, w4v, b4v):
        c = lax.axis_index("c")
        s = lax.axis_index("s")
        w = c * 16 + s
        noff = w * NT32
        pltpu.sync_copy(h1_h.at[pl.ds(noff * 8, NT32 * 8)], h1v)
        pltpu.sync_copy(p0_h.at[pl.ds(noff * 8, NT32 * 8)], p0v)
        pltpu.sync_copy(p1_h.at[pl.ds(noff * 8, NT32 * 8)], p1v)
        pltpu.sync_copy(batch_h.at[pl.ds(noff, NT32)], bv)
        pltpu.sync_copy(w3_h, w3v)
        pltpu.sync_copy(b3_h, b3v)
        pltpu.sync_copy(w4_h, w4v)
        pltpu.sync_copy(b4_h, b4v)
        it = _iota16()
        z16 = jnp.zeros((16,), F32)
        ones16 = jnp.ones((16,), F32)

        def zero_pool(i, carry):
            poolv[pl.ds(i * 16, 16)] = z16
            return carry

        lax.fori_loop(0, GP * 8 // 16, zero_pool, 0)
        for i in range(CW // 16):
            cntv[pl.ds(i * 16, 16)] = z16

        def grp(g, carry):
            rows8 = (g * 16 + it) * 8
            ts = []
            for k in range(8):
                tk = (plsc.load_gather(h1v, [rows8 + k])
                      + plsc.load_gather(p0v, [rows8 + k])
                      + plsc.load_gather(p1v, [rows8 + k]))
                ts.append(tk)
            us = []
            for j in range(8):
                u = b3v[j]
                for k in range(8):
                    u = u + ts[k] * w3v[k, j]
                us.append(jnp.maximum(u, 0.0))
            b16 = bv[pl.ds(g * 16, 16)]
            b8 = b16 * 8
            for j in range(8):
                v = b4v[j]
                for k in range(8):
                    v = v + us[k] * w4v[k, j]
                plsc.addupdate_scatter(poolv, [b8 + j], v)
            plsc.addupdate_scatter(cntv, [b16], ones16)
            return carry

        lax.fori_loop(0, NT32 // 16, grp, 0)
        pltpu.sync_copy(poolv, outp_h.at[pl.ds(w * GP * 8, GP * 8)])
        pltpu.sync_copy(cntv, outc_h.at[pl.ds(w * CW, CW)])

    return k4


@functools.lru_cache(maxsize=None)
def _build_k5():
    @functools.partial(
        pl.kernel,
        out_type=jax.ShapeDtypeStruct((GG,), F32),
        mesh=_mesh(),
        compiler_params=_params(),
        scratch_types=[
            pltpu.VMEM((32 * GP * 8,), F32),
            pltpu.VMEM((32 * CW,), F32),
            pltpu.VMEM((GP * 8,), F32),
            pltpu.VMEM((CW,), F32),
            pltpu.VMEM((8, 16), F32),
            pltpu.VMEM((16,), F32),
            pltpu.VMEM((GG,), F32),
        ],
    )
    def k5(pp_h, pc_h, wfc_h, bfc_h, out_h, ppv, pcv, accp, accc, wfcv,
           bfcv, outv):
        c = lax.axis_index("c")
        s = lax.axis_index("s")

        @pl.when(jnp.logical_and(c == 0, s == 0))
        def _():
            pltpu.sync_copy(pp_h, ppv)
            pltpu.sync_copy(pc_h, pcv)
            pltpu.sync_copy(wfc_h, wfcv)
            pltpu.sync_copy(bfc_h, bfcv)
            it = _iota16()

            def accp_body(i, carry):
                acc = ppv[pl.ds(i * 16, 16)]
                for w in range(1, 32):
                    acc = acc + ppv[pl.ds(w * GP * 8 + i * 16, 16)]
                accp[pl.ds(i * 16, 16)] = acc
                return carry

            lax.fori_loop(0, GP * 8 // 16, accp_body, 0)
            for i in range(GP // 16):
                acc = pcv[pl.ds(i * 16, 16)]
                for w in range(1, 32):
                    acc = acc + pcv[pl.ds(w * CW + i * 16, 16)]
                accc[pl.ds(i * 16, 16)] = acc
            for gg in range(GG // 16):
                cnt = jnp.maximum(accc[pl.ds(gg * 16, 16)], 1.0)
                gidx = (gg * 16 + it) * 8
                z = bfcv[pl.ds(0, 16)]
                for f in range(8):
                    mf = plsc.load_gather(accp, [gidx + f]) / cnt
                    z = z + mf * wfcv[f]
                sig = 1.0 / (1.0 + jnp.exp(-z))
                outv[pl.ds(gg * 16, 16)] = sig
            pltpu.sync_copy(outv, out_h)

    return k5


def kernel(x, edge_index, batch, W1, b1, W2, b2, W3, b3, W4, b4, Wfc, bfc):
    xf = x.reshape(-1).astype(F32)
    xp = jnp.concatenate([xf, jnp.zeros((NP - NN,), F32)])
    src = edge_index[0].astype(I32)
    dst = edge_index[1].astype(I32)
    srcp = jnp.concatenate([src, jnp.zeros((EP - EE,), I32)])
    dstp = jnp.concatenate([dst, jnp.full((EP - EE,), NN, I32)])
    batchp = jnp.concatenate([batch.astype(I32),
                              jnp.full((NP - NN,), GG, I32)])
    z1 = jnp.zeros((NP,), F32)
    z8 = jnp.zeros((NP, 8), F32)
    w1b = jnp.broadcast_to(W1.reshape(8, 1), (8, 16))
    b1b = jnp.broadcast_to(b1.reshape(8, 1), (8, 16))
    w2b = jnp.broadcast_to(W2.reshape(8, 8, 1), (8, 8, 16))
    b2b = jnp.broadcast_to(b2.reshape(8, 1), (8, 16))
    w3b = jnp.broadcast_to(W3.reshape(8, 8, 1), (8, 8, 16))
    b3b = jnp.broadcast_to(b3.reshape(8, 1), (8, 16))
    w4b = jnp.broadcast_to(W4.reshape(8, 8, 1), (8, 8, 16))
    b4b = jnp.broadcast_to(b4.reshape(8, 1), (8, 16))
    wfcb = jnp.broadcast_to(Wfc.reshape(8, 1), (8, 16))
    bfcb = jnp.broadcast_to(bfc.reshape(1), (16,))

    p1a, p1b = _build_edge_phase(1)(xp, srcp, dstp, z1)
    h1f = _build_k2()(xp, p1a, p1b, w1b, b1b, w2b, b2b)
    p2a, p2b = _build_edge_phase(8)(h1f.reshape(NP, 8), srcp, dstp, z8)
    pooled_p, cnt_p = _build_k4()(h1f, p2a.reshape(-1), p2b.reshape(-1),
                                  batchp, w3b, b3b, w4b, b4b)
    out = _build_k5()(pooled_p, cnt_p, wfcb, bfcb)
    return out


# speedup vs baseline: 46.4827x; 1.0423x over previous
import functools

import jax
import jax.numpy as jnp
from jax import lax
from jax.experimental import pallas as pl
from jax.experimental.pallas import tpu as pltpu
from jax.experimental.pallas import tpu_sc as plsc

F32 = jnp.float32
I32 = jnp.int32

NN = 100000
EE = 3200000
GG = 64
GP = 80
CW = 128
NP = 102400
NT32 = NP // 32
NT16 = NP // 16
EP = 3276800
EPT = EP // 32
CH = 3200
NRND = EPT // CH


def _mesh():
    return plsc.VectorSubcoreMesh(core_axis_name="c", subcore_axis_name="s")


def _params():
    return pltpu.CompilerParams(needs_layout_passes=False,
                                use_tc_tiling_on_sc=False)


def _iota16():
    return lax.iota(I32, 16)


@functools.lru_cache(maxsize=None)
def _build_edge_phase(width):
    tshape = (NP,) if width == 1 else (NP, width)
    bshape = (2, CH) if width == 1 else (2, CH, width)

    @functools.partial(
        pl.kernel,
        out_type=(jax.ShapeDtypeStruct(tshape, F32),
                  jax.ShapeDtypeStruct(tshape, F32)),
        mesh=_mesh(),
        compiler_params=_params(),
        scratch_types=[
            pltpu.VMEM((4, CH), I32),
            pltpu.VMEM((4, CH), I32),
            pltpu.VMEM(bshape, F32),
            pltpu.VMEM_SHARED(tshape, F32),
            pltpu.SemaphoreType.DMA((4,)),
            pltpu.SemaphoreType.DMA((4,)),
            pltpu.SemaphoreType.DMA((2,)),
            pltpu.SemaphoreType.DMA((2,)),
        ],
    )
    def kedge(tab_h, src_h, dst_h, z_h, out0_h, out1_h,
              srcb, dstb, gbuf, agg, isem, dsem, gsem, ssem):
        c = lax.axis_index("c")
        s = lax.axis_index("s")
        pltpu.sync_copy(z_h.at[pl.ds(s * NT16, NT16)],
                        agg.at[pl.ds(s * NT16, NT16)])
        plsc.subcore_barrier()
        e0 = (c * 16 + s) * EPT

        def fire_idx(r, sl):
            pltpu.async_copy(src_h.at[pl.ds(e0 + r * CH, CH)],
                             srcb.at[sl], isem.at[sl])
            pltpu.async_copy(dst_h.at[pl.ds(e0 + r * CH, CH)],
                             dstb.at[sl], dsem.at[sl])

        def wait_idx(sl):
            pltpu.make_async_copy(src_h.at[pl.ds(e0, CH)], srcb.at[sl],
                                  isem.at[sl]).wait()
            pltpu.make_async_copy(dst_h.at[pl.ds(e0, CH)], dstb.at[sl],
                                  dsem.at[sl]).wait()

        def wait_gather(b):
            pltpu.make_async_copy(tab_h.at[srcb.at[0]], gbuf.at[b],
                                  gsem.at[b]).wait()

        def fire_scatter(b, sl):
            pltpu.async_copy(gbuf.at[b], agg.at[dstb.at[sl]], ssem.at[b],
                             add=True)

        def wait_scatter(b):
            pltpu.make_async_copy(gbuf.at[b], agg.at[dstb.at[0]],
                                  ssem.at[b]).wait()

        def round_body(r, pos, do_g, do_s, do_pf):
            b, bp = pos % 2, (pos + 1) % 2
            if do_g:
                wait_gather(bp)
                fire_scatter(bp, (pos + 3) % 4)
            if do_s:
                wait_scatter(b)
            if do_pf:
                fire_idx(r + 2, (pos + 2) % 4)
            wait_idx(pos)
            pltpu.async_copy(tab_h.at[srcb.at[pos]], gbuf.at[b], gsem.at[b])

        fire_idx(0, 0)
        fire_idx(1, 1)
        for pos in range(4):
            round_body(pos, pos, pos >= 1, pos >= 2, True)

        def quad(i, carry):
            r = 4 * i
            for pos in range(4):
                round_body(r + pos, pos, True, True, True)
            return carry

        lax.fori_loop(1, NRND // 4 - 1, quad, 0)
        for pos in range(4):
            round_body(NRND - 4 + pos, pos, True, True, pos < 2)
        wait_gather((NRND - 1) % 2)
        fire_scatter((NRND - 1) % 2, (NRND - 1) % 4)
        wait_scatter((NRND - 2) % 2)
        wait_scatter((NRND - 1) % 2)
        plsc.subcore_barrier()
        sl = pl.ds(s * NT16, NT16)

        @pl.when(c == 0)
        def _():
            pltpu.sync_copy(agg.at[sl], out0_h.at[sl])

        @pl.when(c == 1)
        def _():
            pltpu.sync_copy(agg.at[sl], out1_h.at[sl])

    return kedge


@functools.lru_cache(maxsize=None)
def _build_k2():
    @functools.partial(
        pl.kernel,
        out_type=jax.ShapeDtypeStruct((NP * 8,), F32),
        mesh=_mesh(),
        compiler_params=_params(),
        scratch_types=[
            pltpu.VMEM((NT32,), F32),
            pltpu.VMEM((NT32,), F32),
            pltpu.VMEM((NT32,), F32),
            pltpu.VMEM((NT32 * 8,), F32),
            pltpu.VMEM((8, 16), F32),
            pltpu.VMEM((8, 16), F32),
            pltpu.VMEM((8, 8, 16), F32),
            pltpu.VMEM((8, 16), F32),
        ],
    )
    def k2(x_h, p0_h, p1_h, w1_h, b1_h, w2_h, b2_h, out_h,
           xv, p0v, p1v, h1v, w1v, b1v, w2v, b2v):
        c = lax.axis_index("c")
        s = lax.axis_index("s")
        w = c * 16 + s
        noff = w * NT32
        pltpu.sync_copy(x_h.at[pl.ds(noff, NT32)], xv)
        pltpu.sync_copy(p0_h.at[pl.ds(noff, NT32)], p0v)
        pltpu.sync_copy(p1_h.at[pl.ds(noff, NT32)], p1v)
        pltpu.sync_copy(w1_h, w1v)
        pltpu.sync_copy(b1_h, b1v)
        pltpu.sync_copy(w2_h, w2v)
        pltpu.sync_copy(b2_h, b2v)
        it = _iota16()

        def grp(g, carry):
            t = (xv[pl.ds(g * 16, 16)] + p0v[pl.ds(g * 16, 16)]
                 + p1v[pl.ds(g * 16, 16)])
            us = [jnp.maximum(t * w1v[j] + b1v[j], 0.0) for j in range(8)]
            rows8 = (g * 16 + it) * 8
            for j in range(8):
                v = b2v[j]
                for k in range(8):
                    v = v + us[k] * w2v[k, j]
                e = jnp.where(v > 0.0, v, jnp.exp(v) - 1.0)
                plsc.store_scatter(h1v, [rows8 + j], e)
            return carry

        lax.fori_loop(0, NT32 // 16, grp, 0)
        pltpu.sync_copy(h1v, out_h.at[pl.ds(noff * 8, NT32 * 8)])

    return k2


@functools.lru_cache(maxsize=None)
def _build_k4():
    @functools.partial(
        pl.kernel,
        out_type=(jax.ShapeDtypeStruct((32 * GP * 8,), F32),
                  jax.ShapeDtypeStruct((32 * CW,), F32)),
        mesh=_mesh(),
        compiler_params=_params(),
        scratch_types=[
            pltpu.VMEM((NT32 * 8,), F32),
            pltpu.VMEM((NT32 * 8,), F32),
            pltpu.VMEM((NT32 * 8,), F32),
            pltpu.VMEM((NT32,), I32),
            pltpu.VMEM((GP * 8,), F32),
            pltpu.VMEM((CW,), F32),
            pltpu.VMEM((8, 8, 16), F32),
            pltpu.VMEM((8, 16), F32),
            pltpu.VMEM((8, 8, 16), F32),
            pltpu.VMEM((8, 16), F32),
        ],
    )
    def k4(h1_h, p0_h, p1_h, batch_h, w3_h, b3_h, w4_h, b4_h,
           outp_h, outc_h,
           h1v, p0v, p1v, bv, poolv, cntv, w3v, b3v, w4v, b4v):
        c = lax.axis_index("c")
        s = lax.axis_index("s")
        w = c * 16 + s
        noff = w * NT32
        pltpu.sync_copy(h1_h.at[pl.ds(noff * 8, NT32 * 8)], h1v)
        pltpu.sync_copy(p0_h.at[pl.ds(noff * 8, NT32 * 8)], p0v)
        pltpu.sync_copy(p1_h.at[pl.ds(noff * 8, NT32 * 8)], p1v)
        pltpu.sync_copy(batch_h.at[pl.ds(noff, NT32)], bv)
        pltpu.sync_copy(w3_h, w3v)
        pltpu.sync_copy(b3_h, b3v)
        pltpu.sync_copy(w4_h, w4v)
        pltpu.sync_copy(b4_h, b4v)
        it = _iota16()
        z16 = jnp.zeros((16,), F32)
        ones16 = jnp.ones((16,), F32)

        def zero_pool(i, carry):
            poolv[pl.ds(i * 16, 16)] = z16
            return carry

        lax.fori_loop(0, GP * 8 // 16, zero_pool, 0)
        for i in range(CW // 16):
            cntv[pl.ds(i * 16, 16)] = z16

        def grp(g, carry):
            rows8 = (g * 16 + it) * 8
            ts = []
            for k in range(8):
                tk = (plsc.load_gather(h1v, [rows8 + k])
                      + plsc.load_gather(p0v, [rows8 + k])
                      + plsc.load_gather(p1v, [rows8 + k]))
                ts.append(tk)
            us = []
            for j in range(8):
                u = b3v[j]
                for k in range(8):
                    u = u + ts[k] * w3v[k, j]
                us.append(jnp.maximum(u, 0.0))
            b16 = bv[pl.ds(g * 16, 16)]
            b8 = b16 * 8
            for j in range(8):
                v = b4v[j]
                for k in range(8):
                    v = v + us[k] * w4v[k, j]
                plsc.addupdate_scatter(poolv, [b8 + j], v)
            plsc.addupdate_scatter(cntv, [b16], ones16)
            return carry

        lax.fori_loop(0, NT32 // 16, grp, 0)
        pltpu.sync_copy(poolv, outp_h.at[pl.ds(w * GP * 8, GP * 8)])
        pltpu.sync_copy(cntv, outc_h.at[pl.ds(w * CW, CW)])

    return k4


@functools.lru_cache(maxsize=None)
def _build_k5():
    @functools.partial(
        pl.kernel,
        out_type=jax.ShapeDtypeStruct((GG,), F32),
        mesh=_mesh(),
        compiler_params=_params(),
        scratch_types=[
            pltpu.VMEM((32 * GP * 8,), F32),
            pltpu.VMEM((32 * CW,), F32),
            pltpu.VMEM((GP * 8,), F32),
            pltpu.VMEM((CW,), F32),
            pltpu.VMEM((8, 16), F32),
            pltpu.VMEM((16,), F32),
            pltpu.VMEM((GG,), F32),
        ],
    )
    def k5(pp_h, pc_h, wfc_h, bfc_h, out_h, ppv, pcv, accp, accc, wfcv,
           bfcv, outv):
        c = lax.axis_index("c")
        s = lax.axis_index("s")

        @pl.when(jnp.logical_and(c == 0, s == 0))
        def _():
            pltpu.sync_copy(pp_h, ppv)
            pltpu.sync_copy(pc_h, pcv)
            pltpu.sync_copy(wfc_h, wfcv)
            pltpu.sync_copy(bfc_h, bfcv)
            it = _iota16()

            def accp_body(i, carry):
                acc = ppv[pl.ds(i * 16, 16)]
                for w in range(1, 32):
                    acc = acc + ppv[pl.ds(w * GP * 8 + i * 16, 16)]
                accp[pl.ds(i * 16, 16)] = acc
                return carry

            lax.fori_loop(0, GP * 8 // 16, accp_body, 0)
            for i in range(GP // 16):
                acc = pcv[pl.ds(i * 16, 16)]
                for w in range(1, 32):
                    acc = acc + pcv[pl.ds(w * CW + i * 16, 16)]
                accc[pl.ds(i * 16, 16)] = acc
            for gg in range(GG // 16):
                cnt = jnp.maximum(accc[pl.ds(gg * 16, 16)], 1.0)
                gidx = (gg * 16 + it) * 8
                z = bfcv[pl.ds(0, 16)]
                for f in range(8):
                    mf = plsc.load_gather(accp, [gidx + f]) / cnt
                    z = z + mf * wfcv[f]
                sig = 1.0 / (1.0 + jnp.exp(-z))
                outv[pl.ds(gg * 16, 16)] = sig
            pltpu.sync_copy(outv, out_h)

    return k5


def kernel(x, edge_index, batch, W1, b1, W2, b2, W3, b3, W4, b4, Wfc, bfc):
    xf = x.reshape(-1).astype(F32)
    xp = jnp.concatenate([xf, jnp.zeros((NP - NN,), F32)])
    src = edge_index[0].astype(I32)
    dst = edge_index[1].astype(I32)
    srcp = jnp.concatenate([src, jnp.zeros((EP - EE,), I32)])
    dstp = jnp.concatenate([dst, jnp.full((EP - EE,), NN, I32)])
    batchp = jnp.concatenate([batch.astype(I32),
                              jnp.full((NP - NN,), GG, I32)])
    z1 = jnp.zeros((NP,), F32)
    z8 = jnp.zeros((NP, 8), F32)
    w1b = jnp.broadcast_to(W1.reshape(8, 1), (8, 16))
    b1b = jnp.broadcast_to(b1.reshape(8, 1), (8, 16))
    w2b = jnp.broadcast_to(W2.reshape(8, 8, 1), (8, 8, 16))
    b2b = jnp.broadcast_to(b2.reshape(8, 1), (8, 16))
    w3b = jnp.broadcast_to(W3.reshape(8, 8, 1), (8, 8, 16))
    b3b = jnp.broadcast_to(b3.reshape(8, 1), (8, 16))
    w4b = jnp.broadcast_to(W4.reshape(8, 8, 1), (8, 8, 16))
    b4b = jnp.broadcast_to(b4.reshape(8, 1), (8, 16))
    wfcb = jnp.broadcast_to(Wfc.reshape(8, 1), (8, 16))
    bfcb = jnp.broadcast_to(bfc.reshape(1), (16,))

    p1a, p1b = _build_edge_phase(1)(xp, srcp, dstp, z1)
    h1f = _build_k2()(xp, p1a, p1b, w1b, b1b, w2b, b2b)
    p2a, p2b = _build_edge_phase(8)(h1f.reshape(NP, 8), srcp, dstp, z8)
    pooled_p, cnt_p = _build_k4()(h1f, p2a.reshape(-1), p2b.reshape(-1),
                                  batchp, w3b, b3b, w4b, b4b)
    out = _build_k5()(pooled_p, cnt_p, wfcb, bfcb)
    return out


# speedup vs baseline: 49.2899x; 1.0604x over previous
import functools

import jax
import jax.numpy as jnp
from jax import lax
from jax.experimental import pallas as pl
from jax.experimental.pallas import tpu as pltpu
from jax.experimental.pallas import tpu_sc as plsc

F32 = jnp.float32
I32 = jnp.int32

NN = 100000
EE = 3200000
GG = 64
GP = 80
CW = 128
NP = 102400
NT32 = NP // 32
NT16 = NP // 16
EP = 3276800
CH = 3200
F0 = 153600
F1 = 51200
NRND0 = F0 // CH
NRND1 = F1 // CH


def _mesh():
    return plsc.VectorSubcoreMesh(core_axis_name="c", subcore_axis_name="s")


def _params():
    return pltpu.CompilerParams(needs_layout_passes=False,
                                use_tc_tiling_on_sc=False)


def _iota16():
    return lax.iota(I32, 16)


@functools.lru_cache(maxsize=None)
def _build_edge_phase(width):
    tshape = (NP,) if width == 1 else (NP, width)
    bshape = (2, CH) if width == 1 else (2, CH, width)

    @functools.partial(
        pl.kernel,
        out_type=(jax.ShapeDtypeStruct(tshape, F32),
                  jax.ShapeDtypeStruct(tshape, F32)),
        mesh=_mesh(),
        compiler_params=_params(),
        scratch_types=[
            pltpu.VMEM((4, CH), I32),
            pltpu.VMEM((4, CH), I32),
            pltpu.VMEM(bshape, F32),
            pltpu.VMEM_SHARED(tshape, F32),
            pltpu.SemaphoreType.DMA((4,)),
            pltpu.SemaphoreType.DMA((4,)),
            pltpu.SemaphoreType.DMA((2,)),
            pltpu.SemaphoreType.DMA((2,)),
        ],
    )
    def kedge(tab_h, src_h, dst_h, z_h, out0_h, out1_h,
              srcb, dstb, gbuf, agg, isem, dsem, gsem, ssem):
        c = lax.axis_index("c")
        s = lax.axis_index("s")
        pltpu.sync_copy(z_h.at[pl.ds(s * NT16, NT16)],
                        agg.at[pl.ds(s * NT16, NT16)])
        plsc.subcore_barrier()

        def run_edges(e0, nrnd):
            def fire_idx(r, sl):
                pltpu.async_copy(src_h.at[pl.ds(e0 + r * CH, CH)],
                                 srcb.at[sl], isem.at[sl])
                pltpu.async_copy(dst_h.at[pl.ds(e0 + r * CH, CH)],
                                 dstb.at[sl], dsem.at[sl])

            def wait_idx(sl):
                pltpu.make_async_copy(src_h.at[pl.ds(0, CH)], srcb.at[sl],
                                      isem.at[sl]).wait()
                pltpu.make_async_copy(dst_h.at[pl.ds(0, CH)], dstb.at[sl],
                                      dsem.at[sl]).wait()

            def wait_gather(b):
                pltpu.make_async_copy(tab_h.at[srcb.at[0]], gbuf.at[b],
                                      gsem.at[b]).wait()

            def fire_scatter(b, sl):
                pltpu.async_copy(gbuf.at[b], agg.at[dstb.at[sl]],
                                 ssem.at[b], add=True)

            def wait_scatter(b):
                pltpu.make_async_copy(gbuf.at[b], agg.at[dstb.at[0]],
                                      ssem.at[b]).wait()

            def round_body(r, pos, do_g, do_s, do_pf):
                b, bp = pos % 2, (pos + 1) % 2
                if do_g:
                    wait_gather(bp)
                    fire_scatter(bp, (pos + 3) % 4)
                if do_s:
                    wait_scatter(b)
                if do_pf:
                    fire_idx(r + 2, (pos + 2) % 4)
                wait_idx(pos)
                pltpu.async_copy(tab_h.at[srcb.at[pos]], gbuf.at[b],
                                 gsem.at[b])

            fire_idx(0, 0)
            fire_idx(1, 1)
            for pos in range(4):
                round_body(pos, pos, pos >= 1, pos >= 2, True)

            def quad(i, carry):
                r = 4 * i
                for pos in range(4):
                    round_body(r + pos, pos, True, True, True)
                return carry

            lax.fori_loop(1, nrnd // 4 - 1, quad, 0)
            for pos in range(4):
                round_body(nrnd - 4 + pos, pos, True, True, pos < 2)
            wait_gather((nrnd - 1) % 2)
            fire_scatter((nrnd - 1) % 2, (nrnd - 1) % 4)
            wait_scatter((nrnd - 2) % 2)
            wait_scatter((nrnd - 1) % 2)

        @pl.when(c == 0)
        def _():
            run_edges(s * F0, NRND0)

        @pl.when(c == 1)
        def _():
            run_edges(16 * F0 + s * F1, NRND1)

        plsc.subcore_barrier()
        sl = pl.ds(s * NT16, NT16)

        @pl.when(c == 0)
        def _():
            pltpu.sync_copy(agg.at[sl], out0_h.at[sl])

        @pl.when(c == 1)
        def _():
            pltpu.sync_copy(agg.at[sl], out1_h.at[sl])

    return kedge


@functools.lru_cache(maxsize=None)
def _build_k2():
    @functools.partial(
        pl.kernel,
        out_type=jax.ShapeDtypeStruct((NP * 8,), F32),
        mesh=_mesh(),
        compiler_params=_params(),
        scratch_types=[
            pltpu.VMEM((NT32,), F32),
            pltpu.VMEM((NT32,), F32),
            pltpu.VMEM((NT32,), F32),
            pltpu.VMEM((NT32 * 8,), F32),
            pltpu.VMEM((8, 16), F32),
            pltpu.VMEM((8, 16), F32),
            pltpu.VMEM((8, 8, 16), F32),
            pltpu.VMEM((8, 16), F32),
        ],
    )
    def k2(x_h, p0_h, p1_h, w1_h, b1_h, w2_h, b2_h, out_h,
           xv, p0v, p1v, h1v, w1v, b1v, w2v, b2v):
        c = lax.axis_index("c")
        s = lax.axis_index("s")
        w = c * 16 + s
        noff = w * NT32
        pltpu.sync_copy(x_h.at[pl.ds(noff, NT32)], xv)
        pltpu.sync_copy(p0_h.at[pl.ds(noff, NT32)], p0v)
        pltpu.sync_copy(p1_h.at[pl.ds(noff, NT32)], p1v)
        pltpu.sync_copy(w1_h, w1v)
        pltpu.sync_copy(b1_h, b1v)
        pltpu.sync_copy(w2_h, w2v)
        pltpu.sync_copy(b2_h, b2v)
        it = _iota16()

        def grp(g, carry):
            t = (xv[pl.ds(g * 16, 16)] + p0v[pl.ds(g * 16, 16)]
                 + p1v[pl.ds(g * 16, 16)])
            us = [jnp.maximum(t * w1v[j] + b1v[j], 0.0) for j in range(8)]
            rows8 = (g * 16 + it) * 8
            for j in range(8):
                v = b2v[j]
                for k in range(8):
                    v = v + us[k] * w2v[k, j]
                e = jnp.where(v > 0.0, v, jnp.exp(v) - 1.0)
                plsc.store_scatter(h1v, [rows8 + j], e)
            return carry

        lax.fori_loop(0, NT32 // 16, grp, 0)
        pltpu.sync_copy(h1v, out_h.at[pl.ds(noff * 8, NT32 * 8)])

    return k2


@functools.lru_cache(maxsize=None)
def _build_k4():
    @functools.partial(
        pl.kernel,
        out_type=(jax.ShapeDtypeStruct((32 * GP * 8,), F32),
                  jax.ShapeDtypeStruct((32 * CW,), F32)),
        mesh=_mesh(),
        compiler_params=_params(),
        scratch_types=[
            pltpu.VMEM((NT32 * 8,), F32),
            pltpu.VMEM((NT32 * 8,), F32),
            pltpu.VMEM((NT32 * 8,), F32),
            pltpu.VMEM((NT32,), I32),
            pltpu.VMEM((GP * 8,), F32),
            pltpu.VMEM((CW,), F32),
            pltpu.VMEM((8, 8, 16), F32),
            pltpu.VMEM((8, 16), F32),
            pltpu.VMEM((8, 8, 16), F32),
            pltpu.VMEM((8, 16), F32),
        ],
    )
    def k4(h1_h, p0_h, p1_h, batch_h, w3_h, b3_h, w4_h, b4_h,
           outp_h, outc_h,
           h1v, p0v, p1v, bv, poolv, cntv, w3v, b3v, w4v, b4v):
        c = lax.axis_index("c")
        s = lax.axis_index("s")
        w = c * 16 + s
        noff = w * NT32
        pltpu.sync_copy(h1_h.at[pl.ds(noff * 8, NT32 * 8)], h1v)
        pltpu.sync_copy(p0_h.at[pl.ds(noff * 8, NT32 * 8)], p0v)
        pltpu.sync_copy(p1_h.at[pl.ds(noff * 8, NT32 * 8)], p1v)
        pltpu.sync_copy(batch_h.at[pl.ds(noff, NT32)], bv)
        pltpu.sync_copy(w3_h, w3v)
        pltpu.sync_copy(b3_h, b3v)
        pltpu.sync_copy(w4_h, w4v)
        pltpu.sync_copy(b4_h, b4v)
        it = _iota16()
        z16 = jnp.zeros((16,), F32)
        ones16 = jnp.ones((16,), F32)

        def zero_pool(i, carry):
            poolv[pl.ds(i * 16, 16)] = z16
            return carry

        lax.fori_loop(0, GP * 8 // 16, zero_pool, 0)
        for i in range(CW // 16):
            cntv[pl.ds(i * 16, 16)] = z16

        def grp(g, carry):
            rows8 = (g * 16 + it) * 8
            ts = []
            for k in range(8):
                tk = (plsc.load_gather(h1v, [rows8 + k])
                      + plsc.load_gather(p0v, [rows8 + k])
                      + plsc.load_gather(p1v, [rows8 + k]))
                ts.append(tk)
            us = []
            for j in range(8):
                u = b3v[j]
                for k in range(8):
                    u = u + ts[k] * w3v[k, j]
                us.append(jnp.maximum(u, 0.0))
            b16 = bv[pl.ds(g * 16, 16)]
            b8 = b16 * 8
            for j in range(8):
                v = b4v[j]
                for k in range(8):
                    v = v + us[k] * w4v[k, j]
                plsc.addupdate_scatter(poolv, [b8 + j], v)
            plsc.addupdate_scatter(cntv, [b16], ones16)
            return carry

        lax.fori_loop(0, NT32 // 16, grp, 0)
        pltpu.sync_copy(poolv, outp_h.at[pl.ds(w * GP * 8, GP * 8)])
        pltpu.sync_copy(cntv, outc_h.at[pl.ds(w * CW, CW)])

    return k4


@functools.lru_cache(maxsize=None)
def _build_k5():
    @functools.partial(
        pl.kernel,
        out_type=jax.ShapeDtypeStruct((GG,), F32),
        mesh=_mesh(),
        compiler_params=_params(),
        scratch_types=[
            pltpu.VMEM((32 * GP * 8,), F32),
            pltpu.VMEM((32 * CW,), F32),
            pltpu.VMEM((GP * 8,), F32),
            pltpu.VMEM((CW,), F32),
            pltpu.VMEM((8, 16), F32),
            pltpu.VMEM((16,), F32),
            pltpu.VMEM((GG,), F32),
        ],
    )
    def k5(pp_h, pc_h, wfc_h, bfc_h, out_h, ppv, pcv, accp, accc, wfcv,
           bfcv, outv):
        c = lax.axis_index("c")
        s = lax.axis_index("s")

        @pl.when(jnp.logical_and(c == 0, s == 0))
        def _():
            pltpu.sync_copy(pp_h, ppv)
            pltpu.sync_copy(pc_h, pcv)
            pltpu.sync_copy(wfc_h, wfcv)
            pltpu.sync_copy(bfc_h, bfcv)
            it = _iota16()

            def accp_body(i, carry):
                acc = ppv[pl.ds(i * 16, 16)]
                for w in range(1, 32):
                    acc = acc + ppv[pl.ds(w * GP * 8 + i * 16, 16)]
                accp[pl.ds(i * 16, 16)] = acc
                return carry

            lax.fori_loop(0, GP * 8 // 16, accp_body, 0)
            for i in range(GP // 16):
                acc = pcv[pl.ds(i * 16, 16)]
                for w in range(1, 32):
                    acc = acc + pcv[pl.ds(w * CW + i * 16, 16)]
                accc[pl.ds(i * 16, 16)] = acc
            for gg in range(GG // 16):
                cnt = jnp.maximum(accc[pl.ds(gg * 16, 16)], 1.0)
                gidx = (gg * 16 + it) * 8
                z = bfcv[pl.ds(0, 16)]
                for f in range(8):
                    mf = plsc.load_gather(accp, [gidx + f]) / cnt
                    z = z + mf * wfcv[f]
                sig = 1.0 / (1.0 + jnp.exp(-z))
                outv[pl.ds(gg * 16, 16)] = sig
            pltpu.sync_copy(outv, out_h)

    return k5


def kernel(x, edge_index, batch, W1, b1, W2, b2, W3, b3, W4, b4, Wfc, bfc):
    xf = x.reshape(-1).astype(F32)
    xp = jnp.concatenate([xf, jnp.zeros((NP - NN,), F32)])
    src = edge_index[0].astype(I32)
    dst = edge_index[1].astype(I32)
    srcp = jnp.concatenate([src, jnp.zeros((EP - EE,), I32)])
    dstp = jnp.concatenate([dst, jnp.full((EP - EE,), NN, I32)])
    batchp = jnp.concatenate([batch.astype(I32),
                              jnp.full((NP - NN,), GG, I32)])
    z1 = jnp.zeros((NP,), F32)
    z8 = jnp.zeros((NP, 8), F32)
    w1b = jnp.broadcast_to(W1.reshape(8, 1), (8, 16))
    b1b = jnp.broadcast_to(b1.reshape(8, 1), (8, 16))
    w2b = jnp.broadcast_to(W2.reshape(8, 8, 1), (8, 8, 16))
    b2b = jnp.broadcast_to(b2.reshape(8, 1), (8, 16))
    w3b = jnp.broadcast_to(W3.reshape(8, 8, 1), (8, 8, 16))
    b3b = jnp.broadcast_to(b3.reshape(8, 1), (8, 16))
    w4b = jnp.broadcast_to(W4.reshape(8, 8, 1), (8, 8, 16))
    b4b = jnp.broadcast_to(b4.reshape(8, 1), (8, 16))
    wfcb = jnp.broadcast_to(Wfc.reshape(8, 1), (8, 16))
    bfcb = jnp.broadcast_to(bfc.reshape(1), (16,))

    p1a, p1b = _build_edge_phase(1)(xp, srcp, dstp, z1)
    h1f = _build_k2()(xp, p1a, p1b, w1b, b1b, w2b, b2b)
    p2a, p2b = _build_edge_phase(8)(h1f.reshape(NP, 8), srcp, dstp, z8)
    pooled_p, cnt_p = _build_k4()(h1f, p2a.reshape(-1), p2b.reshape(-1),
                                  batchp, w3b, b3b, w4b, b4b)
    out = _build_k5()(pooled_p, cnt_p, wfcb, bfcb)
    return out
